# Initial kernel scaffold; baseline (speedup 1.0000x reference)
import jax
import jax.numpy as jnp
from jax.experimental import pallas as pl


def kernel(node_embeddings, node_to_graph_id, num_graphs, W_s1_mean, b_s1_mean, W_s2_mean, b_s2_mean, W_t1_mean, b_t1_mean, W_t2_mean, b_t2_mean, W_c_mean, W_s1_sum, b_s1_sum, W_s2_sum, b_s2_sum, W_t1_sum, b_t1_sum, W_t2_sum, b_t2_sum, W_c_sum, W_max, W_final):
    raise NotImplementedError("write your pallas kernel here")


# speedup vs baseline: 9.6656x; 9.6656x over previous
import functools

import jax
import jax.numpy as jnp
from jax import lax
from jax.experimental import pallas as pl
from jax.experimental.pallas import tpu as pltpu
from jax.experimental.pallas import tpu_sc as plsc

N, D, H, DH, O, G = 100000, 128, 8, 16, 128, 512
HD = H * DH


NW = 32
C_PER = 3200
RCHUNK = 128
FULL_W = 25
LAST_FULL = 6
TAIL = 32
NEG = float("-inf")


def _sc_segmax_body(x_hbm, idx_hbm, out_hbm, xbuf, table, idxbuf):
    wid = lax.axis_index("s") * 2 + lax.axis_index("c")

    def init(i, _):
        table[pl.ds(i * 16, 16)] = jnp.full((16,), NEG, jnp.float32)
        return 0

    lax.fori_loop(0, (G * D) // 16, init, 0)

    def do_rows(nrows16):
        def group(j, _):
            idx_v = idxbuf[pl.ds(j * 16, 16)]
            for t in range(16):
                g = idx_v[t]
                row = j * 16 + t
                for k in range(D // 16):
                    off = g * D + k * 16
                    cur = table[pl.ds(off, 16)]
                    val = xbuf[pl.ds(row * D + k * 16, 16)]
                    table[pl.ds(off, 16)] = jnp.maximum(cur, val)
            return 0

        lax.fori_loop(0, nrows16, group, 0)

    def chunk(c, _):
        start = wid * C_PER + c * RCHUNK
        pltpu.sync_copy(x_hbm.at[pl.ds(start * D, RCHUNK * D)], xbuf)
        pltpu.sync_copy(idx_hbm.at[pl.ds(start, RCHUNK)],
                        idxbuf.at[pl.ds(0, RCHUNK)])
        do_rows(RCHUNK // 16)
        return 0

    nchunks = jnp.where(wid < NW - 1, FULL_W, LAST_FULL)
    lax.fori_loop(0, nchunks, chunk, 0)

    @pl.when(wid == NW - 1)
    def _tail():
        start = (NW - 1) * C_PER + LAST_FULL * RCHUNK
        pltpu.sync_copy(x_hbm.at[pl.ds(start * D, TAIL * D)],
                        xbuf.at[pl.ds(0, TAIL * D)])
        pltpu.sync_copy(idx_hbm.at[pl.ds(start, TAIL)],
                        idxbuf.at[pl.ds(0, TAIL)])
        do_rows(TAIL // 16)

    pltpu.sync_copy(table, out_hbm.at[pl.ds(wid * G * D, G * D)])


def _sc_segmax(x_flat, idx):
    mesh = plsc.VectorSubcoreMesh(core_axis_name="c", subcore_axis_name="s")
    fn = functools.partial(
        pl.kernel,
        out_type=jax.ShapeDtypeStruct((NW * G * D,), jnp.float32),
        mesh=mesh,
        scratch_types=[
            pltpu.VMEM((RCHUNK * D,), jnp.float32),
            pltpu.VMEM((G * D,), jnp.float32),
            pltpu.VMEM((RCHUNK,), jnp.int32),
        ],
    )(_sc_segmax_body)
    return fn(x_flat, idx)


BBLK = 2000
GRID = N // BBLK


def _expand_mat():
    r = lax.broadcasted_iota(jnp.int32, (H, HD), 0)
    c = lax.broadcasted_iota(jnp.int32, (H, HD), 1)
    return (r == c // DH).astype(jnp.float32)


def _tc_main_body(idx_ref, x_ref,
                  ws1m, bs1m, ws2m, bs2m, wt1m, bt1m, wt2m, bt2m,
                  ws1s, bs1s, ws2s, bs2s, wt1s, bt1s, wt2s, bt2s,
                  a_ref, s_ref, den_ref):
    i = pl.program_id(0)

    @pl.when(i == 0)
    def _init():
        a_ref[...] = jnp.zeros_like(a_ref)
        s_ref[...] = jnp.zeros_like(s_ref)
        den_ref[...] = jnp.zeros_like(den_ref)

    x = x_ref[...]
    f32 = jnp.float32

    def mlp(w1, b1, w2, b2):
        h = jnp.maximum(jnp.dot(x, w1[...], preferred_element_type=f32)
                        + b1[0, :], 0.0)
        return jnp.dot(h, w2[...], preferred_element_type=f32) + b2[0, :]

    ex = jnp.exp(mlp(ws1m, bs1m, ws2m, bs2m))
    sig = jax.nn.sigmoid(mlp(ws1s, bs1s, ws2s, bs2s))
    vm = mlp(wt1m, bt1m, wt2m, bt2m)
    vs = mlp(wt1s, bt1s, wt2s, bt2s)

    e = _expand_mat()
    wm = jnp.dot(ex, e, preferred_element_type=f32) * vm
    ws = jnp.dot(sig, e, preferred_element_type=f32) * vs

    idx = idx_ref[0, 0, :]
    oh = (idx[:, None] == lax.broadcasted_iota(jnp.int32, (BBLK, G), 1)
          ).astype(f32)
    dn = (((0,), (0,)), ((), ()))
    a_ref[...] += lax.dot_general(oh, wm, dn, preferred_element_type=f32)
    s_ref[...] += lax.dot_general(oh, ws, dn, preferred_element_type=f32)
    den_ref[...] += lax.dot_general(oh, ex, dn, preferred_element_type=f32)


def _tc_main(idx3, x, args):
    full = lambda s: pl.BlockSpec(s, lambda i: (0,) * len(s))
    in_specs = [pl.BlockSpec((1, 1, BBLK), lambda i: (i, 0, 0)),
                pl.BlockSpec((BBLK, D), lambda i: (i, 0))]
    in_specs += [full(a.shape) for a in args]
    out_specs = [full((G, HD)), full((G, HD)), full((G, H))]
    out_shape = [jax.ShapeDtypeStruct((G, HD), jnp.float32),
                 jax.ShapeDtypeStruct((G, HD), jnp.float32),
                 jax.ShapeDtypeStruct((G, H), jnp.float32)]
    return pl.pallas_call(
        _tc_main_body,
        grid=(GRID,),
        in_specs=in_specs,
        out_specs=out_specs,
        out_shape=out_shape,
    )(idx3, x, *args)


def _tc_final_body(a_ref, s_ref, den_ref, mx_ref, wcm, wcs, wmax, wf,
                   out_ref):
    f32 = jnp.float32
    e = _expand_mat()
    den = jnp.dot(den_ref[...], e, preferred_element_type=f32) + 1e-16
    mean_r = jnp.dot(a_ref[...] / den, wcm[...], preferred_element_type=f32)
    sum_r = jnp.dot(s_ref[...], wcs[...], preferred_element_type=f32)
    mx = jnp.max(mx_ref[...], axis=0)
    mx = jnp.where(jnp.isfinite(mx), mx, 0.0)
    max_r = jnp.dot(mx, wmax[...], preferred_element_type=f32)
    out = jnp.dot(jnp.maximum(mean_r, 0.0), wf[0],
                  preferred_element_type=f32)
    out += jnp.dot(jnp.maximum(sum_r, 0.0), wf[1],
                   preferred_element_type=f32)
    out += jnp.dot(jnp.maximum(max_r, 0.0), wf[2],
                   preferred_element_type=f32)
    out_ref[...] = out


def _tc_final(a, s, den, mx_parts, wcm, wcs, wmax, wf3):
    return pl.pallas_call(
        _tc_final_body,
        out_shape=jax.ShapeDtypeStruct((G, O), jnp.float32),
    )(a, s, den, mx_parts, wcm, wcs, wmax, wf3)


def kernel(node_embeddings, node_to_graph_id, num_graphs,
           W_s1_mean, b_s1_mean, W_s2_mean, b_s2_mean,
           W_t1_mean, b_t1_mean, W_t2_mean, b_t2_mean, W_c_mean,
           W_s1_sum, b_s1_sum, W_s2_sum, b_s2_sum,
           W_t1_sum, b_t1_sum, W_t2_sum, b_t2_sum, W_c_sum,
           W_max, W_final):
    x = node_embeddings.astype(jnp.float32)
    idx = node_to_graph_id.astype(jnp.int32)

    mx_parts = _sc_segmax(x.reshape(-1), idx).reshape(NW, G, D)

    r2 = lambda b: b.reshape(1, -1)
    args = (W_s1_mean, r2(b_s1_mean), W_s2_mean, r2(b_s2_mean),
            W_t1_mean, r2(b_t1_mean), W_t2_mean, r2(b_t2_mean),
            W_s1_sum, r2(b_s1_sum), W_s2_sum, r2(b_s2_sum),
            W_t1_sum, r2(b_t1_sum), W_t2_sum, r2(b_t2_sum))
    a, s, den = _tc_main(idx.reshape(GRID, 1, BBLK), x, args)

    wf3 = W_final.reshape(3, O, O)
    return _tc_final(a, s, den, mx_parts, W_c_mean, W_c_sum, W_max, wf3)
